# baseline (speedup 1.0000x reference)
import functools

import jax
import jax.numpy as jnp
from jax import lax
from jax.experimental import pallas as pl
from jax.experimental.pallas import tpu as pltpu
from jax.experimental.pallas import tpu_sc as plsc

N_LEVELS = 12
F = 2
LOG2_T = 19
T = 1 << LOG2_T
BASE_RES = 16
SCALE = 1.3819
RES = [int((BASE_RES * SCALE**l) // 1) for l in range(N_LEVELS)]
DENSE = [(r + 1) ** 3 <= T for r in RES]
P1 = 2654435761
P2 = 805459861
CORNERS = [(i, j, k) for i in (0, 1) for j in (0, 1) for k in (0, 1)]

NW = 32
B = 512
GROUPS = B // 16
N_RESIDENT = 2
TAB0_ROWS = -(-((RES[0] + 1) ** 3) // 32) * 32
TAB1_ROWS = -(-((RES[1] + 1) ** 3) // 32) * 32


def _encode(xf, tw2):
    N = xf.shape[0] // 3
    npt = N // NW
    nch = npt // B
    mesh = plsc.VectorSubcoreMesh(core_axis_name="c", subcore_axis_name="s")

    @functools.partial(
        pl.kernel,
        out_type=jax.ShapeDtypeStruct((2 * N_LEVELS, N), jnp.float32),
        mesh=mesh,
        compiler_params=pltpu.CompilerParams(
            needs_layout_passes=False, use_tc_tiling_on_sc=False),
        scratch_types=[
            pltpu.VMEM((3 * B,), jnp.float32),
            pltpu.VMEM((2 * N_LEVELS, B), jnp.float32),
            pltpu.VMEM((8 * B,), jnp.int32),
            pltpu.VMEM((8 * B,), jnp.int32),
            pltpu.VMEM((8 * B, 8), jnp.float32),
            pltpu.VMEM((8 * B, 8), jnp.float32),
            pltpu.VMEM((TAB0_ROWS * 2 // 8, 8), jnp.float32),
            pltpu.VMEM((TAB1_ROWS * 2 // 8, 8), jnp.float32),
            pltpu.SemaphoreType.DMA,
            pltpu.SemaphoreType.DMA,
        ],
    )
    def enc(xf_h, tw8_h, pe_h,
            xb, peb, idxA, idxB, dstA, dstB, tab0, tab1,
            semA, semB):
        wid = lax.axis_index("s") * 2 + lax.axis_index("c")
        idxb = (idxA, idxB)
        dstb = (dstA, dstB)
        sems = (semA, semB)

        pltpu.sync_copy(tw8_h.at[pl.ds(0, TAB0_ROWS * 2 // 8)], tab0)
        pltpu.sync_copy(tw8_h.at[pl.ds(T // 4, TAB1_ROWS * 2 // 8)], tab1)

        iota16 = lax.iota(jnp.int32, 16)
        zeros16 = jnp.zeros((16,), jnp.int32)
        ones16 = jnp.full((16,), 1, jnp.int32)

        def coords(g, l):
            res = RES[l]
            resf = float(res)
            p3 = (g * 16 + iota16) * 3
            fx = plsc.load_gather(xb, [p3]) * resf
            fy = plsc.load_gather(xb, [p3 + 1]) * resf
            fz = plsc.load_gather(xb, [p3 + 2]) * resf
            ix = jnp.clip(fx.astype(jnp.int32), 0, res - 1)
            iy = jnp.clip(fy.astype(jnp.int32), 0, res - 1)
            iz = jnp.clip(fz.astype(jnp.int32), 0, res - 1)
            return fx, fy, fz, ix, iy, iz

        def corner_rows(l, ix, iy, iz):
            res = RES[l]
            if DENSE[l]:
                stride = res + 1
                ax = (ix, ix + 1)
                ay = (iy * stride, iy * stride + stride)
                az = (iz * (stride * stride), iz * (stride * stride) + stride * stride)
                return [ax[i] + ay[j] + az[k] for (i, j, k) in CORNERS]
            ux, uy, uz = (ix.astype(jnp.uint32), iy.astype(jnp.uint32),
                          iz.astype(jnp.uint32))
            hx = (ux, ux + jnp.uint32(1))
            hy = (uy * jnp.uint32(P1), uy * jnp.uint32(P1) + jnp.uint32(P1))
            hz = (uz * jnp.uint32(P2), uz * jnp.uint32(P2) + jnp.uint32(P2))
            return [((hx[i] ^ hy[j] ^ hz[k]) & jnp.uint32(T - 1)).astype(jnp.int32)
                    for (i, j, k) in CORNERS]

        def weights(fx, fy, fz, ix, iy, iz):
            wx = fx - ix.astype(jnp.float32)
            wy = fy - iy.astype(jnp.float32)
            wz = fz - iz.astype(jnp.float32)
            u = (1.0 - wx, wx)
            v = (1.0 - wy, wy)
            t = (1.0 - wz, wz)
            pxy = [u[i] * v[j] for i in (0, 1) for j in (0, 1)]
            return pxy, t

        def chunk_body(ch, carry):
            base = wid * npt + ch * B
            pltpu.sync_copy(xf_h.at[pl.ds(base * 3, B * 3)], xb)

            def idx_loop(l):
                lbase = l * T
                bi = l % 2

                def body(g, c, l=l, lbase=lbase, bi=bi):
                    _, _, _, ix, iy, iz = coords(g, l)
                    rows = corner_rows(l, ix, iy, iz)
                    for ci in range(8):
                        idxb[bi][pl.ds(ci * B + g * 16, 16)] = (rows[ci] + lbase) >> 2
                    return c

                lax.fori_loop(0, GROUPS, body, 0)

            def fire(l):
                bi = l % 2
                return pltpu.async_copy(tw8_h.at[idxb[bi]], dstb[bi], sems[bi])

            def acc_loop(l):
                bi = l % 2

                def body(g, c, l=l, bi=bi):
                    fx, fy, fz, ix, iy, iz = coords(g, l)
                    rows = corner_rows(l, ix, iy, iz)
                    pxy, t = weights(fx, fy, fz, ix, iy, iz)
                    prow = g * 16 + iota16
                    acc0 = acc1 = None
                    for ci, (i, j, k) in enumerate(CORNERS):
                        wc = pxy[2 * i + j] * t[k]
                        ocol = (rows[ci] & 3) * 2
                        v0 = plsc.load_gather(dstb[bi], [prow + ci * B, ocol])
                        v1 = plsc.load_gather(dstb[bi], [prow + ci * B, ocol + 1])
                        if acc0 is None:
                            acc0, acc1 = wc * v0, wc * v1
                        else:
                            acc0, acc1 = acc0 + wc * v0, acc1 + wc * v1
                    peb[2 * l, pl.ds(g * 16, 16)] = acc0
                    peb[2 * l + 1, pl.ds(g * 16, 16)] = acc1
                    return c

                lax.fori_loop(0, GROUPS, body, 0)

            def resident_loop(l, tab):
                def body(g, c, l=l):
                    fx, fy, fz, ix, iy, iz = coords(g, l)
                    rows = corner_rows(l, ix, iy, iz)
                    pxy, t = weights(fx, fy, fz, ix, iy, iz)
                    acc0 = acc1 = None
                    for ci, (i, j, k) in enumerate(CORNERS):
                        wc = pxy[2 * i + j] * t[k]
                        r8 = rows[ci] >> 2
                        oc = (rows[ci] & 3) * 2
                        v0 = plsc.load_gather(tab, [r8, oc])
                        v1 = plsc.load_gather(tab, [r8, oc + 1])
                        if acc0 is None:
                            acc0, acc1 = wc * v0, wc * v1
                        else:
                            acc0, acc1 = acc0 + wc * v0, acc1 + wc * v1
                    peb[2 * l, pl.ds(g * 16, 16)] = acc0
                    peb[2 * l + 1, pl.ds(g * 16, 16)] = acc1
                    return c

                lax.fori_loop(0, GROUPS, body, 0)

            idx_loop(2)
            d_prev = fire(2)
            resident_loop(0, tab0)
            resident_loop(1, tab1)
            for l in range(3, N_LEVELS):
                idx_loop(l)
                d_next = fire(l)
                d_prev.wait()
                acc_loop(l - 1)
                d_prev = d_next
            d_prev.wait()
            acc_loop(N_LEVELS - 1)

            pltpu.sync_copy(peb, pe_h.at[:, pl.ds(base, B)])
            return carry

        lax.fori_loop(0, nch, chunk_body, 0)

    return enc(xf, tw2.reshape(-1, 8))


def _mlp(pe_t, W0, W1, W2):
    N = pe_t.shape[1]
    BN = 1024
    dn0 = (((0,), (0,)), ((), ()))
    dn = (((1,), (0,)), ((), ()))

    def body(pe_ref, w0_ref, w1_ref, w2_ref, pe_out_ref, z_ref, d_ref):
        pt = pe_ref[...]
        pe_out_ref[...] = pt.T
        h = jnp.maximum(
            lax.dot_general(pt, w0_ref[...], dn0, preferred_element_type=jnp.float32), 0.0)
        h = jnp.maximum(
            lax.dot_general(h, w1_ref[...], dn, preferred_element_type=jnp.float32), 0.0)
        z = lax.dot_general(h, w2_ref[...], dn, preferred_element_type=jnp.float32)
        z_ref[...] = z
        z0 = z[:, 0:1]
        d_ref[...] = jnp.maximum(z0, 0.0) + jnp.log1p(jnp.exp(-jnp.abs(z0)))

    pe, z, dens = pl.pallas_call(
        body,
        grid=(N // BN,),
        in_specs=[
            pl.BlockSpec((2 * N_LEVELS, BN), lambda i: (0, i)),
            pl.BlockSpec((2 * N_LEVELS, 64), lambda i: (0, 0)),
            pl.BlockSpec((64, 64), lambda i: (0, 0)),
            pl.BlockSpec((64, 16), lambda i: (0, 0)),
        ],
        out_specs=[
            pl.BlockSpec((BN, 2 * N_LEVELS), lambda i: (i, 0)),
            pl.BlockSpec((BN, 16), lambda i: (i, 0)),
            pl.BlockSpec((BN, 1), lambda i: (i, 0)),
        ],
        out_shape=[
            jax.ShapeDtypeStruct((N, 2 * N_LEVELS), jnp.float32),
            jax.ShapeDtypeStruct((N, 16), jnp.float32),
            jax.ShapeDtypeStruct((N, 1), jnp.float32),
        ],
    )(pe_t, W0, W1, W2)
    return pe, z, dens


def kernel(x, tables, W0, W1, W2):
    N = x.shape[0]
    xnf = (x * jnp.float32(1.0 / 256.0)).reshape(-1)
    pe_t = _encode(xnf, tables.reshape(N_LEVELS * T, F))
    pe, z, dens = _mlp(pe_t, W0, W1, W2)
    return (dens.reshape(x.shape[:-1]), pe, z)


# speedup vs baseline: 2.1689x; 2.1689x over previous
import functools

import jax
import jax.numpy as jnp
from jax import lax
from jax.experimental import pallas as pl
from jax.experimental.pallas import tpu as pltpu
from jax.experimental.pallas import tpu_sc as plsc

N_LEVELS = 12
F = 2
LOG2_T = 19
T = 1 << LOG2_T
BASE_RES = 16
SCALE = 1.3819
RES = [int((BASE_RES * SCALE**l) // 1) for l in range(N_LEVELS)]
DENSE = [(r + 1) ** 3 <= T for r in RES]
P1 = 2654435761
P2 = 805459861
CORNERS = [(i, j, k) for i in (0, 1) for j in (0, 1) for k in (0, 1)]

NW = 32
B = 512
GROUPS = B // 16
N_RESIDENT = 2
TAB0_ROWS = -(-((RES[0] + 1) ** 3) // 32) * 32
TAB1_ROWS = -(-((RES[1] + 1) ** 3) // 32) * 32


NBLK = N_LEVELS * (T // 128)
BLK_PER_TILE = NBLK // NW
CB = 32


def _relayout(tn8):
    R8 = N_LEVELS * T * 2 // 8
    mesh = plsc.VectorSubcoreMesh(core_axis_name="c", subcore_axis_name="s")

    @functools.partial(
        pl.kernel,
        out_type=jax.ShapeDtypeStruct((R8, 8), jnp.float32),
        mesh=mesh,
        compiler_params=pltpu.CompilerParams(
            needs_layout_passes=False, use_tc_tiling_on_sc=False),
        scratch_types=[
            pltpu.VMEM((CB * 32, 8), jnp.float32),
            pltpu.VMEM((CB * 32, 8), jnp.float32),
        ],
    )
    def rel(tn_h, out_h, ibuf, obuf):
        wid = lax.axis_index("s") * 2 + lax.axis_index("c")
        iota16 = lax.iota(jnp.int32, 16)
        riota = iota16 >> 3
        ciota = iota16 & 7
        ors = [(2 * iota16 + f) >> 3 for f in (0, 1)]
        ocs = [(2 * iota16 + f) & 7 for f in (0, 1)]

        def chunk_body(ch, carry):
            row0 = (wid * BLK_PER_TILE + ch * CB) * 32
            pltpu.sync_copy(tn_h.at[pl.ds(row0, CB * 32)], ibuf)

            def blk_body(b, c):
                for j in range(8):
                    for f in (0, 1):
                        rsrc = b * 32 + f * 16 + 2 * j + riota
                        v = plsc.load_gather(ibuf, [rsrc, ciota])
                        rdst = b * 32 + 4 * j + ors[f]
                        plsc.store_scatter(obuf, [rdst, ocs[f]], v)
                return c

            lax.fori_loop(0, CB, blk_body, 0)
            pltpu.sync_copy(obuf, out_h.at[pl.ds(row0, CB * 32)])
            return carry

        lax.fori_loop(0, BLK_PER_TILE // CB, chunk_body, 0)

    return rel(tn8)


def _encode(xf, tw8):
    N = xf.shape[0] // 3
    npt = N // NW
    nch = npt // B
    mesh = plsc.VectorSubcoreMesh(core_axis_name="c", subcore_axis_name="s")

    @functools.partial(
        pl.kernel,
        out_type=jax.ShapeDtypeStruct((2 * N_LEVELS, N), jnp.float32),
        mesh=mesh,
        compiler_params=pltpu.CompilerParams(
            needs_layout_passes=False, use_tc_tiling_on_sc=False),
        scratch_types=[
            pltpu.VMEM((3 * B,), jnp.float32),
            pltpu.VMEM((2 * N_LEVELS, B), jnp.float32),
            pltpu.VMEM((8 * B,), jnp.int32),
            pltpu.VMEM((8 * B,), jnp.int32),
            pltpu.VMEM((8 * B, 8), jnp.float32),
            pltpu.VMEM((8 * B, 8), jnp.float32),
            pltpu.VMEM((TAB0_ROWS * 2 // 8, 8), jnp.float32),
            pltpu.VMEM((TAB1_ROWS * 2 // 8, 8), jnp.float32),
            pltpu.SemaphoreType.DMA,
            pltpu.SemaphoreType.DMA,
        ],
    )
    def enc(xf_h, tw8_h, pe_h,
            xb, peb, idxA, idxB, dstA, dstB, tab0, tab1,
            semA, semB):
        wid = lax.axis_index("s") * 2 + lax.axis_index("c")
        idxb = (idxA, idxB)
        dstb = (dstA, dstB)
        sems = (semA, semB)

        pltpu.sync_copy(tw8_h.at[pl.ds(0, TAB0_ROWS * 2 // 8)], tab0)
        pltpu.sync_copy(tw8_h.at[pl.ds(T // 4, TAB1_ROWS * 2 // 8)], tab1)

        iota16 = lax.iota(jnp.int32, 16)
        zeros16 = jnp.zeros((16,), jnp.int32)
        ones16 = jnp.full((16,), 1, jnp.int32)

        def coords(g, l):
            res = RES[l]
            resf = float(res)
            p3 = (g * 16 + iota16) * 3
            fx = plsc.load_gather(xb, [p3]) * resf
            fy = plsc.load_gather(xb, [p3 + 1]) * resf
            fz = plsc.load_gather(xb, [p3 + 2]) * resf
            ix = jnp.clip(fx.astype(jnp.int32), 0, res - 1)
            iy = jnp.clip(fy.astype(jnp.int32), 0, res - 1)
            iz = jnp.clip(fz.astype(jnp.int32), 0, res - 1)
            return fx, fy, fz, ix, iy, iz

        def corner_rows(l, ix, iy, iz):
            res = RES[l]
            if DENSE[l]:
                stride = res + 1
                ax = (ix, ix + 1)
                ay = (iy * stride, iy * stride + stride)
                az = (iz * (stride * stride), iz * (stride * stride) + stride * stride)
                return [ax[i] + ay[j] + az[k] for (i, j, k) in CORNERS]
            ux, uy, uz = (ix.astype(jnp.uint32), iy.astype(jnp.uint32),
                          iz.astype(jnp.uint32))
            hx = (ux, ux + jnp.uint32(1))
            hy = (uy * jnp.uint32(P1), uy * jnp.uint32(P1) + jnp.uint32(P1))
            hz = (uz * jnp.uint32(P2), uz * jnp.uint32(P2) + jnp.uint32(P2))
            return [((hx[i] ^ hy[j] ^ hz[k]) & jnp.uint32(T - 1)).astype(jnp.int32)
                    for (i, j, k) in CORNERS]

        def weights(fx, fy, fz, ix, iy, iz):
            wx = fx - ix.astype(jnp.float32)
            wy = fy - iy.astype(jnp.float32)
            wz = fz - iz.astype(jnp.float32)
            u = (1.0 - wx, wx)
            v = (1.0 - wy, wy)
            t = (1.0 - wz, wz)
            pxy = [u[i] * v[j] for i in (0, 1) for j in (0, 1)]
            return pxy, t

        def chunk_body(ch, carry):
            base = wid * npt + ch * B
            pltpu.sync_copy(xf_h.at[pl.ds(base * 3, B * 3)], xb)

            def idx_loop(l):
                lbase = l * T
                bi = l % 2

                def body(g, c, l=l, lbase=lbase, bi=bi):
                    _, _, _, ix, iy, iz = coords(g, l)
                    rows = corner_rows(l, ix, iy, iz)
                    for ci in range(8):
                        idxb[bi][pl.ds(ci * B + g * 16, 16)] = (rows[ci] + lbase) >> 2
                    return c

                lax.fori_loop(0, GROUPS, body, 0)

            def fire(l):
                bi = l % 2
                return pltpu.async_copy(tw8_h.at[idxb[bi]], dstb[bi], sems[bi])

            def acc_loop(l):
                bi = l % 2

                def body(g, c, l=l, bi=bi):
                    fx, fy, fz, ix, iy, iz = coords(g, l)
                    rows = corner_rows(l, ix, iy, iz)
                    pxy, t = weights(fx, fy, fz, ix, iy, iz)
                    prow = g * 16 + iota16
                    acc0 = acc1 = None
                    for ci, (i, j, k) in enumerate(CORNERS):
                        wc = pxy[2 * i + j] * t[k]
                        ocol = (rows[ci] & 3) * 2
                        v0 = plsc.load_gather(dstb[bi], [prow + ci * B, ocol])
                        v1 = plsc.load_gather(dstb[bi], [prow + ci * B, ocol + 1])
                        if acc0 is None:
                            acc0, acc1 = wc * v0, wc * v1
                        else:
                            acc0, acc1 = acc0 + wc * v0, acc1 + wc * v1
                    peb[2 * l, pl.ds(g * 16, 16)] = acc0
                    peb[2 * l + 1, pl.ds(g * 16, 16)] = acc1
                    return c

                lax.fori_loop(0, GROUPS, body, 0)

            def resident_loop(l, tab):
                def body(g, c, l=l):
                    fx, fy, fz, ix, iy, iz = coords(g, l)
                    rows = corner_rows(l, ix, iy, iz)
                    pxy, t = weights(fx, fy, fz, ix, iy, iz)
                    acc0 = acc1 = None
                    for ci, (i, j, k) in enumerate(CORNERS):
                        wc = pxy[2 * i + j] * t[k]
                        r8 = rows[ci] >> 2
                        oc = (rows[ci] & 3) * 2
                        v0 = plsc.load_gather(tab, [r8, oc])
                        v1 = plsc.load_gather(tab, [r8, oc + 1])
                        if acc0 is None:
                            acc0, acc1 = wc * v0, wc * v1
                        else:
                            acc0, acc1 = acc0 + wc * v0, acc1 + wc * v1
                    peb[2 * l, pl.ds(g * 16, 16)] = acc0
                    peb[2 * l + 1, pl.ds(g * 16, 16)] = acc1
                    return c

                lax.fori_loop(0, GROUPS, body, 0)

            idx_loop(2)
            d_prev = fire(2)
            resident_loop(0, tab0)
            resident_loop(1, tab1)
            for l in range(3, N_LEVELS):
                idx_loop(l)
                d_next = fire(l)
                d_prev.wait()
                acc_loop(l - 1)
                d_prev = d_next
            d_prev.wait()
            acc_loop(N_LEVELS - 1)

            pltpu.sync_copy(peb, pe_h.at[:, pl.ds(base, B)])
            return carry

        lax.fori_loop(0, nch, chunk_body, 0)

    return enc(xf, tw8)


def _mlp(pe_t, W0, W1, W2):
    N = pe_t.shape[1]
    BN = 1024
    dn0 = (((0,), (0,)), ((), ()))
    dn = (((1,), (0,)), ((), ()))

    def body(pe_ref, w0_ref, w1_ref, w2_ref, pe_out_ref, z_ref, d_ref):
        pt = pe_ref[...]
        pe_out_ref[...] = pt
        h = jnp.maximum(
            lax.dot_general(pt, w0_ref[...], dn0, preferred_element_type=jnp.float32), 0.0)
        h = jnp.maximum(
            lax.dot_general(h, w1_ref[...], dn, preferred_element_type=jnp.float32), 0.0)
        zt = lax.dot_general(w2_ref[...], h, (((0,), (1,)), ((), ())),
                             preferred_element_type=jnp.float32)
        z_ref[...] = zt
        z0 = zt[0:1, :]
        d_ref[...] = jnp.maximum(z0, 0.0) + jnp.log1p(jnp.exp(-jnp.abs(z0)))

    pe_c, z_t, dens = pl.pallas_call(
        body,
        grid=(N // BN,),
        in_specs=[
            pl.BlockSpec((2 * N_LEVELS, BN), lambda i: (0, i)),
            pl.BlockSpec((2 * N_LEVELS, 64), lambda i: (0, 0)),
            pl.BlockSpec((64, 64), lambda i: (0, 0)),
            pl.BlockSpec((64, 16), lambda i: (0, 0)),
        ],
        out_specs=[
            pl.BlockSpec((2 * N_LEVELS, BN), lambda i: (0, i)),
            pl.BlockSpec((16, BN), lambda i: (0, i)),
            pl.BlockSpec((1, BN), lambda i: (0, i)),
        ],
        out_shape=[
            jax.ShapeDtypeStruct((2 * N_LEVELS, N), jnp.float32),
            jax.ShapeDtypeStruct((16, N), jnp.float32),
            jax.ShapeDtypeStruct((1, N), jnp.float32),
        ],
    )(pe_t, W0, W1, W2)
    return pe_c, z_t, dens


def kernel(x, tables, W0, W1, W2):
    N = x.shape[0]
    xnf = (x * jnp.float32(1.0 / 256.0)).reshape(-1)
    tn8 = tables.reshape(N_LEVELS, T // 128, 128, F).transpose(0, 1, 3, 2).reshape(-1, 8)
    tw8 = _relayout(tn8)
    pe_t = _encode(xnf, tw8)
    pe_c, z_t, dens = _mlp(pe_t, W0, W1, W2)
    return (dens.reshape(x.shape[:-1]), pe_c.T, z_t.T)


# speedup vs baseline: 2.1741x; 1.0024x over previous
import functools

import jax
import jax.numpy as jnp
from jax import lax
from jax.experimental import pallas as pl
from jax.experimental.pallas import tpu as pltpu
from jax.experimental.pallas import tpu_sc as plsc

N_LEVELS = 12
F = 2
LOG2_T = 19
T = 1 << LOG2_T
BASE_RES = 16
SCALE = 1.3819
RES = [int((BASE_RES * SCALE**l) // 1) for l in range(N_LEVELS)]
DENSE = [(r + 1) ** 3 <= T for r in RES]
P1 = 2654435761
P2 = 805459861
CORNERS = [(i, j, k) for i in (0, 1) for j in (0, 1) for k in (0, 1)]

NW = 32
B = 512
GROUPS = B // 16
N_RESIDENT = 2
TAB0_ROWS = -(-((RES[0] + 1) ** 3) // 32) * 32
TAB1_ROWS = -(-((RES[1] + 1) ** 3) // 32) * 32


NBLK = N_LEVELS * (T // 128)
BLK_PER_TILE = NBLK // NW
CB = 32


def _relayout(tn8):
    R8 = N_LEVELS * T * 2 // 8
    mesh = plsc.VectorSubcoreMesh(core_axis_name="c", subcore_axis_name="s")

    @functools.partial(
        pl.kernel,
        out_type=jax.ShapeDtypeStruct((R8, 8), jnp.float32),
        mesh=mesh,
        compiler_params=pltpu.CompilerParams(
            needs_layout_passes=False, use_tc_tiling_on_sc=False),
        scratch_types=[
            pltpu.VMEM((CB * 32, 8), jnp.float32),
            pltpu.VMEM((CB * 32, 8), jnp.float32),
        ],
    )
    def rel(tn_h, out_h, ibuf, obuf):
        wid = lax.axis_index("s") * 2 + lax.axis_index("c")
        iota16 = lax.iota(jnp.int32, 16)
        riota = iota16 >> 3
        ciota = iota16 & 7
        ors = [(2 * iota16 + f) >> 3 for f in (0, 1)]
        ocs = [(2 * iota16 + f) & 7 for f in (0, 1)]

        def chunk_body(ch, carry):
            row0 = (wid * BLK_PER_TILE + ch * CB) * 32
            pltpu.sync_copy(tn_h.at[pl.ds(row0, CB * 32)], ibuf)

            def blk_body(b, c):
                for j in range(8):
                    for f in (0, 1):
                        rsrc = b * 32 + f * 16 + 2 * j + riota
                        v = plsc.load_gather(ibuf, [rsrc, ciota])
                        rdst = b * 32 + 4 * j + ors[f]
                        plsc.store_scatter(obuf, [rdst, ocs[f]], v)
                return c

            lax.fori_loop(0, CB, blk_body, 0)
            pltpu.sync_copy(obuf, out_h.at[pl.ds(row0, CB * 32)])
            return carry

        lax.fori_loop(0, BLK_PER_TILE // CB, chunk_body, 0)

    return rel(tn8)


def _encode(x, tw8):
    N = x.shape[0]
    npt = N // NW
    nch = npt // B
    mesh = plsc.VectorSubcoreMesh(core_axis_name="c", subcore_axis_name="s")

    @functools.partial(
        pl.kernel,
        out_type=jax.ShapeDtypeStruct((2 * N_LEVELS, N), jnp.float32),
        mesh=mesh,
        compiler_params=pltpu.CompilerParams(
            needs_layout_passes=False, use_tc_tiling_on_sc=False),
        scratch_types=[
            pltpu.VMEM((B, 3), jnp.float32),
            pltpu.VMEM((2 * N_LEVELS, B), jnp.float32),
            pltpu.VMEM((8 * B,), jnp.int32),
            pltpu.VMEM((8 * B,), jnp.int32),
            pltpu.VMEM((8 * B, 8), jnp.float32),
            pltpu.VMEM((8 * B, 8), jnp.float32),
            pltpu.VMEM((TAB0_ROWS * 2 // 8, 8), jnp.float32),
            pltpu.VMEM((TAB1_ROWS * 2 // 8, 8), jnp.float32),
            pltpu.SemaphoreType.DMA,
            pltpu.SemaphoreType.DMA,
        ],
    )
    def enc(x_h, tw8_h, pe_h,
            xb, peb, idxA, idxB, dstA, dstB, tab0, tab1,
            semA, semB):
        wid = lax.axis_index("s") * 2 + lax.axis_index("c")
        idxb = (idxA, idxB)
        dstb = (dstA, dstB)
        sems = (semA, semB)

        pltpu.sync_copy(tw8_h.at[pl.ds(0, TAB0_ROWS * 2 // 8)], tab0)
        pltpu.sync_copy(tw8_h.at[pl.ds(T // 4, TAB1_ROWS * 2 // 8)], tab1)

        iota16 = lax.iota(jnp.int32, 16)
        zeros16 = jnp.zeros((16,), jnp.int32)
        ones16 = jnp.full((16,), 1, jnp.int32)

        def coords(g, l):
            res = RES[l]
            resf = float(res)
            p16 = g * 16 + iota16
            fx = plsc.load_gather(xb, [p16, zeros16]) * resf
            fy = plsc.load_gather(xb, [p16, ones16]) * resf
            fz = plsc.load_gather(xb, [p16, ones16 + 1]) * resf
            ix = jnp.clip(fx.astype(jnp.int32), 0, res - 1)
            iy = jnp.clip(fy.astype(jnp.int32), 0, res - 1)
            iz = jnp.clip(fz.astype(jnp.int32), 0, res - 1)
            return fx, fy, fz, ix, iy, iz

        def corner_rows(l, ix, iy, iz):
            res = RES[l]
            if DENSE[l]:
                stride = res + 1
                ax = (ix, ix + 1)
                ay = (iy * stride, iy * stride + stride)
                az = (iz * (stride * stride), iz * (stride * stride) + stride * stride)
                return [ax[i] + ay[j] + az[k] for (i, j, k) in CORNERS]
            ux, uy, uz = (ix.astype(jnp.uint32), iy.astype(jnp.uint32),
                          iz.astype(jnp.uint32))
            hx = (ux, ux + jnp.uint32(1))
            hy = (uy * jnp.uint32(P1), uy * jnp.uint32(P1) + jnp.uint32(P1))
            hz = (uz * jnp.uint32(P2), uz * jnp.uint32(P2) + jnp.uint32(P2))
            return [((hx[i] ^ hy[j] ^ hz[k]) & jnp.uint32(T - 1)).astype(jnp.int32)
                    for (i, j, k) in CORNERS]

        def weights(fx, fy, fz, ix, iy, iz):
            wx = fx - ix.astype(jnp.float32)
            wy = fy - iy.astype(jnp.float32)
            wz = fz - iz.astype(jnp.float32)
            u = (1.0 - wx, wx)
            v = (1.0 - wy, wy)
            t = (1.0 - wz, wz)
            pxy = [u[i] * v[j] for i in (0, 1) for j in (0, 1)]
            return pxy, t

        def chunk_body(ch, carry):
            base = wid * npt + ch * B
            pltpu.sync_copy(x_h.at[pl.ds(base, B)], xb)

            def norm_body(g, c):
                p16 = g * 16 + iota16
                for d in range(3):
                    dv = jnp.full((16,), d, jnp.int32)
                    v = plsc.load_gather(xb, [p16, dv])
                    plsc.store_scatter(xb, [p16, dv], v * jnp.float32(1.0 / 256.0))
                return c

            lax.fori_loop(0, GROUPS, norm_body, 0)

            def idx_loop(l):
                lbase = l * T
                bi = l % 2

                def body(g, c, l=l, lbase=lbase, bi=bi):
                    _, _, _, ix, iy, iz = coords(g, l)
                    rows = corner_rows(l, ix, iy, iz)
                    for ci in range(8):
                        idxb[bi][pl.ds(ci * B + g * 16, 16)] = (rows[ci] + lbase) >> 2
                    return c

                lax.fori_loop(0, GROUPS, body, 0)

            def fire(l):
                bi = l % 2
                return pltpu.async_copy(tw8_h.at[idxb[bi]], dstb[bi], sems[bi])

            def acc_loop(l):
                bi = l % 2

                def body(g, c, l=l, bi=bi):
                    fx, fy, fz, ix, iy, iz = coords(g, l)
                    rows = corner_rows(l, ix, iy, iz)
                    pxy, t = weights(fx, fy, fz, ix, iy, iz)
                    prow = g * 16 + iota16
                    acc0 = acc1 = None
                    for ci, (i, j, k) in enumerate(CORNERS):
                        wc = pxy[2 * i + j] * t[k]
                        ocol = (rows[ci] & 3) * 2
                        v0 = plsc.load_gather(dstb[bi], [prow + ci * B, ocol])
                        v1 = plsc.load_gather(dstb[bi], [prow + ci * B, ocol + 1])
                        if acc0 is None:
                            acc0, acc1 = wc * v0, wc * v1
                        else:
                            acc0, acc1 = acc0 + wc * v0, acc1 + wc * v1
                    peb[2 * l, pl.ds(g * 16, 16)] = acc0
                    peb[2 * l + 1, pl.ds(g * 16, 16)] = acc1
                    return c

                lax.fori_loop(0, GROUPS, body, 0)

            def resident_loop(l, tab):
                def body(g, c, l=l):
                    fx, fy, fz, ix, iy, iz = coords(g, l)
                    rows = corner_rows(l, ix, iy, iz)
                    pxy, t = weights(fx, fy, fz, ix, iy, iz)
                    acc0 = acc1 = None
                    for ci, (i, j, k) in enumerate(CORNERS):
                        wc = pxy[2 * i + j] * t[k]
                        r8 = rows[ci] >> 2
                        oc = (rows[ci] & 3) * 2
                        v0 = plsc.load_gather(tab, [r8, oc])
                        v1 = plsc.load_gather(tab, [r8, oc + 1])
                        if acc0 is None:
                            acc0, acc1 = wc * v0, wc * v1
                        else:
                            acc0, acc1 = acc0 + wc * v0, acc1 + wc * v1
                    peb[2 * l, pl.ds(g * 16, 16)] = acc0
                    peb[2 * l + 1, pl.ds(g * 16, 16)] = acc1
                    return c

                lax.fori_loop(0, GROUPS, body, 0)

            idx_loop(2)
            d_prev = fire(2)
            resident_loop(0, tab0)
            resident_loop(1, tab1)
            for l in range(3, N_LEVELS):
                idx_loop(l)
                d_next = fire(l)
                d_prev.wait()
                acc_loop(l - 1)
                d_prev = d_next
            d_prev.wait()
            acc_loop(N_LEVELS - 1)

            pltpu.sync_copy(peb, pe_h.at[:, pl.ds(base, B)])
            return carry

        lax.fori_loop(0, nch, chunk_body, 0)

    return enc(x, tw8)


def _mlp(pe_t, W0, W1, W2):
    N = pe_t.shape[1]
    BN = 2048
    dn0 = (((0,), (0,)), ((), ()))
    dn = (((1,), (0,)), ((), ()))

    def body(pe_ref, w0_ref, w1_ref, w2_ref, pe_out_ref, z_ref, d_ref):
        pt = pe_ref[...]
        pe_out_ref[...] = pt
        h = jnp.maximum(
            lax.dot_general(pt, w0_ref[...], dn0, preferred_element_type=jnp.float32), 0.0)
        h = jnp.maximum(
            lax.dot_general(h, w1_ref[...], dn, preferred_element_type=jnp.float32), 0.0)
        zt = lax.dot_general(w2_ref[...], h, (((0,), (1,)), ((), ())),
                             preferred_element_type=jnp.float32)
        z_ref[...] = zt
        z0 = zt[0:1, :]
        d_ref[...] = jnp.maximum(z0, 0.0) + jnp.log1p(jnp.exp(-jnp.abs(z0)))

    pe_c, z_t, dens = pl.pallas_call(
        body,
        grid=(N // BN,),
        in_specs=[
            pl.BlockSpec((2 * N_LEVELS, BN), lambda i: (0, i)),
            pl.BlockSpec((2 * N_LEVELS, 64), lambda i: (0, 0)),
            pl.BlockSpec((64, 64), lambda i: (0, 0)),
            pl.BlockSpec((64, 16), lambda i: (0, 0)),
        ],
        out_specs=[
            pl.BlockSpec((2 * N_LEVELS, BN), lambda i: (0, i)),
            pl.BlockSpec((16, BN), lambda i: (0, i)),
            pl.BlockSpec((1, BN), lambda i: (0, i)),
        ],
        out_shape=[
            jax.ShapeDtypeStruct((2 * N_LEVELS, N), jnp.float32),
            jax.ShapeDtypeStruct((16, N), jnp.float32),
            jax.ShapeDtypeStruct((1, N), jnp.float32),
        ],
    )(pe_t, W0, W1, W2)
    return pe_c, z_t, dens


def kernel(x, tables, W0, W1, W2):
    N = x.shape[0]
    tn8 = tables.reshape(N_LEVELS, T // 128, 128, F).transpose(0, 1, 3, 2).reshape(-1, 8)
    tw8 = _relayout(tn8)
    pe_t = _encode(x, tw8)
    pe_c, z_t, dens = _mlp(pe_t, W0, W1, W2)
    return (dens.reshape(x.shape[:-1]), pe_c.T, z_t.T)


# speedup vs baseline: 3.0456x; 1.4008x over previous
import functools

import jax
import jax.numpy as jnp
from jax import lax
from jax.experimental import pallas as pl
from jax.experimental.pallas import tpu as pltpu
from jax.experimental.pallas import tpu_sc as plsc

N_LEVELS = 12
F = 2
LOG2_T = 19
T = 1 << LOG2_T
BASE_RES = 16
SCALE = 1.3819
RES = [int((BASE_RES * SCALE**l) // 1) for l in range(N_LEVELS)]
DENSE = [(r + 1) ** 3 <= T for r in RES]
P1 = 2654435761
P2 = 805459861
CORNERS = [(i, j, k) for i in (0, 1) for j in (0, 1) for k in (0, 1)]

NW = 32
B = 512
GROUPS = B // 16
N_RESIDENT = 2
TAB0_ROWS = -(-((RES[0] + 1) ** 3) // 32) * 32
TAB1_ROWS = -(-((RES[1] + 1) ** 3) // 32) * 32


NBLK = N_LEVELS * (T // 128)
BLK_PER_TILE = NBLK // NW
CB = 32


def _relayout(tn8):
    R8 = N_LEVELS * T * 2 // 8
    mesh = plsc.VectorSubcoreMesh(core_axis_name="c", subcore_axis_name="s")

    @functools.partial(
        pl.kernel,
        out_type=jax.ShapeDtypeStruct((R8, 8), jnp.float32),
        mesh=mesh,
        compiler_params=pltpu.CompilerParams(
            needs_layout_passes=False, use_tc_tiling_on_sc=False),
        scratch_types=[
            pltpu.VMEM((CB * 32, 8), jnp.float32),
            pltpu.VMEM((CB * 32, 8), jnp.float32),
        ],
    )
    def rel(tn_h, out_h, ibuf, obuf):
        wid = lax.axis_index("s") * 2 + lax.axis_index("c")
        iota16 = lax.iota(jnp.int32, 16)
        riota = iota16 >> 3
        ciota = iota16 & 7
        ors = [(2 * iota16 + f) >> 3 for f in (0, 1)]
        ocs = [(2 * iota16 + f) & 7 for f in (0, 1)]

        def chunk_body(ch, carry):
            row0 = (wid * BLK_PER_TILE + ch * CB) * 32
            pltpu.sync_copy(tn_h.at[pl.ds(row0, CB * 32)], ibuf)

            def blk_body(b, c):
                for j in range(8):
                    for f in (0, 1):
                        rsrc = b * 32 + f * 16 + 2 * j + riota
                        v = plsc.load_gather(ibuf, [rsrc, ciota])
                        rdst = b * 32 + 4 * j + ors[f]
                        plsc.store_scatter(obuf, [rdst, ocs[f]], v)
                return c

            lax.fori_loop(0, CB, blk_body, 0)
            pltpu.sync_copy(obuf, out_h.at[pl.ds(row0, CB * 32)])
            return carry

        lax.fori_loop(0, BLK_PER_TILE // CB, chunk_body, 0)

    return rel(tn8)


def _encode(x, tw8):
    N = x.shape[0]
    npt = N // NW
    nch = npt // B
    mesh = plsc.VectorSubcoreMesh(core_axis_name="c", subcore_axis_name="s")

    @functools.partial(
        pl.kernel,
        out_type=jax.ShapeDtypeStruct((3, N // 128, 8, 128), jnp.float32),
        mesh=mesh,
        compiler_params=pltpu.CompilerParams(
            needs_layout_passes=False, use_tc_tiling_on_sc=False),
        scratch_types=[
            pltpu.VMEM((B, 3), jnp.float32),
            pltpu.VMEM((3, B // 128, 8, 128), jnp.float32),
            pltpu.VMEM((8 * B,), jnp.int32),
            pltpu.VMEM((8 * B,), jnp.int32),
            pltpu.VMEM((8 * B, 8), jnp.float32),
            pltpu.VMEM((8 * B, 8), jnp.float32),
            pltpu.VMEM((TAB0_ROWS * 2 // 8, 8), jnp.float32),
            pltpu.VMEM((TAB1_ROWS * 2 // 8, 8), jnp.float32),
            pltpu.SemaphoreType.DMA,
            pltpu.SemaphoreType.DMA,
        ],
    )
    def enc(x_h, tw8_h, pe_h,
            xb, peb, idxA, idxB, dstA, dstB, tab0, tab1,
            semA, semB):
        wid = lax.axis_index("s") * 2 + lax.axis_index("c")
        idxb = (idxA, idxB)
        dstb = (dstA, dstB)
        sems = (semA, semB)

        pltpu.sync_copy(tw8_h.at[pl.ds(0, TAB0_ROWS * 2 // 8)], tab0)
        pltpu.sync_copy(tw8_h.at[pl.ds(T // 4, TAB1_ROWS * 2 // 8)], tab1)

        iota16 = lax.iota(jnp.int32, 16)
        zeros16 = jnp.zeros((16,), jnp.int32)
        ones16 = jnp.full((16,), 1, jnp.int32)

        def coords(g, l):
            res = RES[l]
            resf = float(res)
            p16 = g * 16 + iota16
            fx = plsc.load_gather(xb, [p16, zeros16]) * resf
            fy = plsc.load_gather(xb, [p16, ones16]) * resf
            fz = plsc.load_gather(xb, [p16, ones16 + 1]) * resf
            ix = jnp.clip(fx.astype(jnp.int32), 0, res - 1)
            iy = jnp.clip(fy.astype(jnp.int32), 0, res - 1)
            iz = jnp.clip(fz.astype(jnp.int32), 0, res - 1)
            return fx, fy, fz, ix, iy, iz

        def corner_rows(l, ix, iy, iz):
            res = RES[l]
            if DENSE[l]:
                stride = res + 1
                ax = (ix, ix + 1)
                ay = (iy * stride, iy * stride + stride)
                az = (iz * (stride * stride), iz * (stride * stride) + stride * stride)
                return [ax[i] + ay[j] + az[k] for (i, j, k) in CORNERS]
            ux, uy, uz = (ix.astype(jnp.uint32), iy.astype(jnp.uint32),
                          iz.astype(jnp.uint32))
            hx = (ux, ux + jnp.uint32(1))
            hy = (uy * jnp.uint32(P1), uy * jnp.uint32(P1) + jnp.uint32(P1))
            hz = (uz * jnp.uint32(P2), uz * jnp.uint32(P2) + jnp.uint32(P2))
            return [((hx[i] ^ hy[j] ^ hz[k]) & jnp.uint32(T - 1)).astype(jnp.int32)
                    for (i, j, k) in CORNERS]

        def weights(fx, fy, fz, ix, iy, iz):
            wx = fx - ix.astype(jnp.float32)
            wy = fy - iy.astype(jnp.float32)
            wz = fz - iz.astype(jnp.float32)
            u = (1.0 - wx, wx)
            v = (1.0 - wy, wy)
            t = (1.0 - wz, wz)
            pxy = [u[i] * v[j] for i in (0, 1) for j in (0, 1)]
            return pxy, t

        def chunk_body(ch, carry):
            base = wid * npt + ch * B
            pltpu.sync_copy(x_h.at[pl.ds(base, B)], xb)

            def norm_body(g, c):
                p16 = g * 16 + iota16
                for d in range(3):
                    dv = jnp.full((16,), d, jnp.int32)
                    v = plsc.load_gather(xb, [p16, dv])
                    plsc.store_scatter(xb, [p16, dv], v * jnp.float32(1.0 / 256.0))
                return c

            lax.fori_loop(0, GROUPS, norm_body, 0)

            def idx_loop(l):
                lbase = l * T
                bi = l % 2

                def body(g, c, l=l, lbase=lbase, bi=bi):
                    _, _, _, ix, iy, iz = coords(g, l)
                    rows = corner_rows(l, ix, iy, iz)
                    for ci in range(8):
                        idxb[bi][pl.ds(ci * B + g * 16, 16)] = (rows[ci] + lbase) >> 2
                    return c

                lax.fori_loop(0, GROUPS, body, 0)

            def fire(l):
                bi = l % 2
                return pltpu.async_copy(tw8_h.at[idxb[bi]], dstb[bi], sems[bi])

            def acc_loop(l):
                bi = l % 2

                def body(g, c, l=l, bi=bi):
                    fx, fy, fz, ix, iy, iz = coords(g, l)
                    rows = corner_rows(l, ix, iy, iz)
                    pxy, t = weights(fx, fy, fz, ix, iy, iz)
                    prow = g * 16 + iota16
                    acc0 = acc1 = None
                    for ci, (i, j, k) in enumerate(CORNERS):
                        wc = pxy[2 * i + j] * t[k]
                        ocol = (rows[ci] & 3) * 2
                        v0 = plsc.load_gather(dstb[bi], [prow + ci * B, ocol])
                        v1 = plsc.load_gather(dstb[bi], [prow + ci * B, ocol + 1])
                        if acc0 is None:
                            acc0, acc1 = wc * v0, wc * v1
                        else:
                            acc0, acc1 = acc0 + wc * v0, acc1 + wc * v1
                    _pe_store(l, g, acc0, acc1)
                    return c

                lax.fori_loop(0, GROUPS, body, 0)

            def _pe_store(l, g, acc0, acc1):
                ct = g >> 3
                col = (g & 7) * 16
                for f, acc in ((0, acc0), (1, acc1)):
                    r = 2 * l + f
                    peb[r >> 3, ct, r & 7, pl.ds(col, 16)] = acc

            def resident_loop(l, tab):
                def body(g, c, l=l):
                    fx, fy, fz, ix, iy, iz = coords(g, l)
                    rows = corner_rows(l, ix, iy, iz)
                    pxy, t = weights(fx, fy, fz, ix, iy, iz)
                    acc0 = acc1 = None
                    for ci, (i, j, k) in enumerate(CORNERS):
                        wc = pxy[2 * i + j] * t[k]
                        r8 = rows[ci] >> 2
                        oc = (rows[ci] & 3) * 2
                        v0 = plsc.load_gather(tab, [r8, oc])
                        v1 = plsc.load_gather(tab, [r8, oc + 1])
                        if acc0 is None:
                            acc0, acc1 = wc * v0, wc * v1
                        else:
                            acc0, acc1 = acc0 + wc * v0, acc1 + wc * v1
                    _pe_store(l, g, acc0, acc1)
                    return c

                lax.fori_loop(0, GROUPS, body, 0)

            idx_loop(2)
            d_prev = fire(2)
            resident_loop(0, tab0)
            resident_loop(1, tab1)
            for l in range(3, N_LEVELS):
                idx_loop(l)
                d_next = fire(l)
                d_prev.wait()
                acc_loop(l - 1)
                d_prev = d_next
            d_prev.wait()
            acc_loop(N_LEVELS - 1)

            ct0 = base // 128
            for i in range(3):
                pltpu.sync_copy(peb.at[i], pe_h.at[i, pl.ds(ct0, B // 128)])
            return carry

        lax.fori_loop(0, nch, chunk_body, 0)

    return enc(x, tw8)


def _mlp(pe_t, W0, W1, W2):
    N = pe_t.shape[1]
    BN = 2048
    dn0 = (((0,), (0,)), ((), ()))
    dn = (((1,), (0,)), ((), ()))

    def body(pe_ref, w0_ref, w1_ref, w2_ref, pe_out_ref, z_ref, d_ref):
        pt = pe_ref[...]
        pe_out_ref[...] = pt
        h = jnp.maximum(
            lax.dot_general(pt, w0_ref[...], dn0, preferred_element_type=jnp.float32), 0.0)
        h = jnp.maximum(
            lax.dot_general(h, w1_ref[...], dn, preferred_element_type=jnp.float32), 0.0)
        zt = lax.dot_general(w2_ref[...], h, (((0,), (1,)), ((), ())),
                             preferred_element_type=jnp.float32)
        z_ref[...] = zt
        z0 = zt[0:1, :]
        d_ref[...] = (jnp.maximum(z0, 0.0)
                      + jnp.log1p(jnp.exp(-jnp.abs(z0)))).reshape(-1)

    pe_c, z_t, dens = pl.pallas_call(
        body,
        grid=(N // BN,),
        in_specs=[
            pl.BlockSpec((2 * N_LEVELS, BN), lambda i: (0, i)),
            pl.BlockSpec((2 * N_LEVELS, 64), lambda i: (0, 0)),
            pl.BlockSpec((64, 64), lambda i: (0, 0)),
            pl.BlockSpec((64, 16), lambda i: (0, 0)),
        ],
        out_specs=[
            pl.BlockSpec((2 * N_LEVELS, BN), lambda i: (0, i)),
            pl.BlockSpec((16, BN), lambda i: (0, i)),
            pl.BlockSpec((BN,), lambda i: (i,)),
        ],
        out_shape=[
            jax.ShapeDtypeStruct((2 * N_LEVELS, N), jnp.float32),
            jax.ShapeDtypeStruct((16, N), jnp.float32),
            jax.ShapeDtypeStruct((N,), jnp.float32),
        ],
    )(pe_t, W0, W1, W2)
    return pe_c, z_t, dens


def kernel(x, tables, W0, W1, W2):
    N = x.shape[0]
    tn8 = tables.reshape(N_LEVELS, T // 128, 128, F).transpose(0, 1, 3, 2).reshape(-1, 8)
    tw8 = _relayout(tn8)
    pe4 = _encode(x, tw8)
    pe_t = pe4.transpose(0, 2, 1, 3).reshape(2 * N_LEVELS, N)
    pe_c, z_t, dens = _mlp(pe_t, W0, W1, W2)
    return (dens, pe_c.T, z_t.T)


# speedup vs baseline: 4.3069x; 1.4142x over previous
import functools

import jax
import jax.numpy as jnp
from jax import lax
from jax.experimental import pallas as pl
from jax.experimental.pallas import tpu as pltpu
from jax.experimental.pallas import tpu_sc as plsc

N_LEVELS = 12
F = 2
LOG2_T = 19
T = 1 << LOG2_T
BASE_RES = 16
SCALE = 1.3819
RES = [int((BASE_RES * SCALE**l) // 1) for l in range(N_LEVELS)]
DENSE = [(r + 1) ** 3 <= T for r in RES]
P1 = 2654435761
P2 = 805459861
CORNERS = [(i, j, k) for i in (0, 1) for j in (0, 1) for k in (0, 1)]

NW = 32
B = 512
GROUPS = B // 16
N_RESIDENT = 2
TAB0_ROWS = -(-((RES[0] + 1) ** 3) // 32) * 32
TAB1_ROWS = -(-((RES[1] + 1) ** 3) // 32) * 32


NBLK = N_LEVELS * (T // 128)
BLK_PER_TILE = NBLK // NW
CB = 32


def _relayout(tn8):
    R8 = N_LEVELS * T * 2 // 8
    mesh = plsc.VectorSubcoreMesh(core_axis_name="c", subcore_axis_name="s")

    @functools.partial(
        pl.kernel,
        out_type=jax.ShapeDtypeStruct((R8, 8), jnp.float32),
        mesh=mesh,
        compiler_params=pltpu.CompilerParams(
            needs_layout_passes=False, use_tc_tiling_on_sc=False),
        scratch_types=[
            pltpu.VMEM((CB * 32, 8), jnp.float32),
            pltpu.VMEM((CB * 32, 8), jnp.float32),
        ],
    )
    def rel(tn_h, out_h, ibuf, obuf):
        wid = lax.axis_index("s") * 2 + lax.axis_index("c")
        iota16 = lax.iota(jnp.int32, 16)
        riota = iota16 >> 3
        ciota = iota16 & 7
        ors = [(2 * iota16 + f) >> 3 for f in (0, 1)]
        ocs = [(2 * iota16 + f) & 7 for f in (0, 1)]

        def chunk_body(ch, carry):
            row0 = (wid * BLK_PER_TILE + ch * CB) * 32
            pltpu.sync_copy(tn_h.at[pl.ds(row0, CB * 32)], ibuf)

            def blk_body(b, c):
                for j in range(8):
                    for f in (0, 1):
                        rsrc = b * 32 + f * 16 + 2 * j + riota
                        v = plsc.load_gather(ibuf, [rsrc, ciota])
                        rdst = b * 32 + 4 * j + ors[f]
                        plsc.store_scatter(obuf, [rdst, ocs[f]], v)
                return c

            lax.fori_loop(0, CB, blk_body, 0)
            pltpu.sync_copy(obuf, out_h.at[pl.ds(row0, CB * 32)])
            return carry

        lax.fori_loop(0, BLK_PER_TILE // CB, chunk_body, 0)

    return rel(tn8)


def _encode(xt, tw8):
    N = xt.shape[1]
    npt = N // NW
    nch = npt // B
    mesh = plsc.VectorSubcoreMesh(core_axis_name="c", subcore_axis_name="s")

    @functools.partial(
        pl.kernel,
        out_type=jax.ShapeDtypeStruct((3, N // 128, 8, 128), jnp.float32),
        mesh=mesh,
        compiler_params=pltpu.CompilerParams(
            needs_layout_passes=False, use_tc_tiling_on_sc=False),
        scratch_types=[
            pltpu.VMEM((3, B), jnp.float32),
            pltpu.VMEM((3, B // 128, 8, 128), jnp.float32),
            pltpu.VMEM((8 * B,), jnp.int32),
            pltpu.VMEM((8 * B,), jnp.int32),
            pltpu.VMEM((8 * B, 8), jnp.float32),
            pltpu.VMEM((8 * B, 8), jnp.float32),
            pltpu.VMEM((TAB0_ROWS * 2 // 8, 8), jnp.float32),
            pltpu.VMEM((TAB1_ROWS * 2 // 8, 8), jnp.float32),
            pltpu.SemaphoreType.DMA,
            pltpu.SemaphoreType.DMA,
        ],
    )
    def enc(xt_h, tw8_h, pe_h,
            xb, peb, idxA, idxB, dstA, dstB, tab0, tab1,
            semA, semB):
        wid = lax.axis_index("s") * 2 + lax.axis_index("c")
        idxb = (idxA, idxB)
        dstb = (dstA, dstB)
        sems = (semA, semB)

        pltpu.sync_copy(tw8_h.at[pl.ds(0, TAB0_ROWS * 2 // 8)], tab0)
        pltpu.sync_copy(tw8_h.at[pl.ds(T // 4, TAB1_ROWS * 2 // 8)], tab1)

        iota16 = lax.iota(jnp.int32, 16)
        zeros16 = jnp.zeros((16,), jnp.int32)
        ones16 = jnp.full((16,), 1, jnp.int32)

        def coords(g, l):
            res = RES[l]
            resf = float(res)
            s = pl.ds(g * 16, 16)
            fx = xb[0, s] * resf
            fy = xb[1, s] * resf
            fz = xb[2, s] * resf
            ix = jnp.clip(fx.astype(jnp.int32), 0, res - 1)
            iy = jnp.clip(fy.astype(jnp.int32), 0, res - 1)
            iz = jnp.clip(fz.astype(jnp.int32), 0, res - 1)
            return fx, fy, fz, ix, iy, iz

        def corner_rows(l, ix, iy, iz):
            res = RES[l]
            if DENSE[l]:
                stride = res + 1
                ax = (ix, ix + 1)
                ay = (iy * stride, iy * stride + stride)
                az = (iz * (stride * stride), iz * (stride * stride) + stride * stride)
                return [ax[i] + ay[j] + az[k] for (i, j, k) in CORNERS]
            ux, uy, uz = (ix.astype(jnp.uint32), iy.astype(jnp.uint32),
                          iz.astype(jnp.uint32))
            hx = (ux, ux + jnp.uint32(1))
            hy = (uy * jnp.uint32(P1), uy * jnp.uint32(P1) + jnp.uint32(P1))
            hz = (uz * jnp.uint32(P2), uz * jnp.uint32(P2) + jnp.uint32(P2))
            return [((hx[i] ^ hy[j] ^ hz[k]) & jnp.uint32(T - 1)).astype(jnp.int32)
                    for (i, j, k) in CORNERS]

        def weights(fx, fy, fz, ix, iy, iz):
            wx = fx - ix.astype(jnp.float32)
            wy = fy - iy.astype(jnp.float32)
            wz = fz - iz.astype(jnp.float32)
            u = (1.0 - wx, wx)
            v = (1.0 - wy, wy)
            t = (1.0 - wz, wz)
            pxy = [u[i] * v[j] for i in (0, 1) for j in (0, 1)]
            return pxy, t

        def chunk_body(ch, carry):
            base = wid * npt + ch * B
            pltpu.sync_copy(xt_h.at[:, pl.ds(base, B)], xb)

            def norm_body(g, c):
                s = pl.ds(g * 16, 16)
                for d in range(3):
                    xb[d, s] = xb[d, s] * jnp.float32(1.0 / 256.0)
                return c

            lax.fori_loop(0, GROUPS, norm_body, 0)

            def idx_loop(l):
                lbase = l * T
                bi = l % 2

                def body(g, c, l=l, lbase=lbase, bi=bi):
                    _, _, _, ix, iy, iz = coords(g, l)
                    rows = corner_rows(l, ix, iy, iz)
                    for ci in range(8):
                        idxb[bi][pl.ds(ci * B + g * 16, 16)] = (rows[ci] + lbase) >> 2
                    return c

                lax.fori_loop(0, GROUPS, body, 0)

            def fire(l):
                bi = l % 2
                return pltpu.async_copy(tw8_h.at[idxb[bi]], dstb[bi], sems[bi])

            def acc_loop(l):
                bi = l % 2

                def body(g, c, l=l, bi=bi):
                    fx, fy, fz, ix, iy, iz = coords(g, l)
                    rows = corner_rows(l, ix, iy, iz)
                    pxy, t = weights(fx, fy, fz, ix, iy, iz)
                    prow = g * 16 + iota16
                    acc0 = acc1 = None
                    for ci, (i, j, k) in enumerate(CORNERS):
                        wc = pxy[2 * i + j] * t[k]
                        ocol = (rows[ci] & 3) * 2
                        v0 = plsc.load_gather(dstb[bi], [prow + ci * B, ocol])
                        v1 = plsc.load_gather(dstb[bi], [prow + ci * B, ocol + 1])
                        if acc0 is None:
                            acc0, acc1 = wc * v0, wc * v1
                        else:
                            acc0, acc1 = acc0 + wc * v0, acc1 + wc * v1
                    _pe_store(l, g, acc0, acc1)
                    return c

                lax.fori_loop(0, GROUPS, body, 0)

            def _pe_store(l, g, acc0, acc1):
                ct = g >> 3
                col = (g & 7) * 16
                for f, acc in ((0, acc0), (1, acc1)):
                    r = 2 * l + f
                    peb[r >> 3, ct, r & 7, pl.ds(col, 16)] = acc

            def resident_loop(l, tab):
                def body(g, c, l=l):
                    fx, fy, fz, ix, iy, iz = coords(g, l)
                    rows = corner_rows(l, ix, iy, iz)
                    pxy, t = weights(fx, fy, fz, ix, iy, iz)
                    acc0 = acc1 = None
                    for ci, (i, j, k) in enumerate(CORNERS):
                        wc = pxy[2 * i + j] * t[k]
                        r8 = rows[ci] >> 2
                        oc = (rows[ci] & 3) * 2
                        v0 = plsc.load_gather(tab, [r8, oc])
                        v1 = plsc.load_gather(tab, [r8, oc + 1])
                        if acc0 is None:
                            acc0, acc1 = wc * v0, wc * v1
                        else:
                            acc0, acc1 = acc0 + wc * v0, acc1 + wc * v1
                    _pe_store(l, g, acc0, acc1)
                    return c

                lax.fori_loop(0, GROUPS, body, 0)

            idx_loop(2)
            d_prev = fire(2)
            resident_loop(0, tab0)
            resident_loop(1, tab1)
            for l in range(3, N_LEVELS):
                idx_loop(l)
                d_next = fire(l)
                d_prev.wait()
                acc_loop(l - 1)
                d_prev = d_next
            d_prev.wait()
            acc_loop(N_LEVELS - 1)

            ct0 = base // 128
            for i in range(3):
                pltpu.sync_copy(peb.at[i], pe_h.at[i, pl.ds(ct0, B // 128)])
            return carry

        lax.fori_loop(0, nch, chunk_body, 0)

    return enc(xt, tw8)


def _mlp(pe_t, W0, W1, W2):
    N = pe_t.shape[1]
    BN = 2048
    dn0 = (((0,), (0,)), ((), ()))
    dn = (((1,), (0,)), ((), ()))

    def body(pe_ref, w0_ref, w1_ref, w2_ref, pe_out_ref, z_ref, d_ref):
        pt = pe_ref[...]
        pe_out_ref[...] = pt
        h = jnp.maximum(
            lax.dot_general(pt, w0_ref[...], dn0, preferred_element_type=jnp.float32), 0.0)
        h = jnp.maximum(
            lax.dot_general(h, w1_ref[...], dn, preferred_element_type=jnp.float32), 0.0)
        zt = lax.dot_general(w2_ref[...], h, (((0,), (1,)), ((), ())),
                             preferred_element_type=jnp.float32)
        z_ref[...] = zt
        z0 = zt[0:1, :]
        d_ref[...] = (jnp.maximum(z0, 0.0)
                      + jnp.log1p(jnp.exp(-jnp.abs(z0)))).reshape(-1)

    pe_c, z_t, dens = pl.pallas_call(
        body,
        grid=(N // BN,),
        in_specs=[
            pl.BlockSpec((2 * N_LEVELS, BN), lambda i: (0, i)),
            pl.BlockSpec((2 * N_LEVELS, 64), lambda i: (0, 0)),
            pl.BlockSpec((64, 64), lambda i: (0, 0)),
            pl.BlockSpec((64, 16), lambda i: (0, 0)),
        ],
        out_specs=[
            pl.BlockSpec((2 * N_LEVELS, BN), lambda i: (0, i)),
            pl.BlockSpec((16, BN), lambda i: (0, i)),
            pl.BlockSpec((BN,), lambda i: (i,)),
        ],
        out_shape=[
            jax.ShapeDtypeStruct((2 * N_LEVELS, N), jnp.float32),
            jax.ShapeDtypeStruct((16, N), jnp.float32),
            jax.ShapeDtypeStruct((N,), jnp.float32),
        ],
    )(pe_t, W0, W1, W2)
    return pe_c, z_t, dens


def kernel(x, tables, W0, W1, W2):
    N = x.shape[0]
    tn8 = tables.reshape(N_LEVELS, T // 128, 128, F).transpose(0, 1, 3, 2).reshape(-1, 8)
    tw8 = _relayout(tn8)
    pe4 = _encode(x.T, tw8)
    pe_t = pe4.transpose(0, 2, 1, 3).reshape(2 * N_LEVELS, N)
    pe_c, z_t, dens = _mlp(pe_t, W0, W1, W2)
    return (dens, pe_c.T, z_t.T)


# speedup vs baseline: 4.6367x; 1.0766x over previous
import functools

import jax
import jax.numpy as jnp
from jax import lax
from jax.experimental import pallas as pl
from jax.experimental.pallas import tpu as pltpu
from jax.experimental.pallas import tpu_sc as plsc

N_LEVELS = 12
F = 2
LOG2_T = 19
T = 1 << LOG2_T
BASE_RES = 16
SCALE = 1.3819
RES = [int((BASE_RES * SCALE**l) // 1) for l in range(N_LEVELS)]
DENSE = [(r + 1) ** 3 <= T for r in RES]
P1 = 2654435761
P2 = 805459861
CORNERS = [(i, j, k) for i in (0, 1) for j in (0, 1) for k in (0, 1)]

NW = 32
SPM_LEVELS = (2,)
SPM_CNT = [-(-(((RES[l] + 1) ** 3 + 3) // 4) // 8) * 8 for l in SPM_LEVELS]
SPM_OFF = {2: 0}
SPM_TOT = sum(SPM_CNT)
B = 512
GROUPS = B // 16
N_RESIDENT = 2
TAB0_ROWS = -(-((RES[0] + 1) ** 3) // 32) * 32
TAB1_ROWS = -(-((RES[1] + 1) ** 3) // 32) * 32


NBLK = N_LEVELS * (T // 128)
BLK_PER_TILE = NBLK // NW
CB = 32


def _relayout(tn8):
    R8 = N_LEVELS * T * 2 // 8
    mesh = plsc.VectorSubcoreMesh(core_axis_name="c", subcore_axis_name="s")

    @functools.partial(
        pl.kernel,
        out_type=jax.ShapeDtypeStruct((R8, 8), jnp.float32),
        mesh=mesh,
        compiler_params=pltpu.CompilerParams(
            needs_layout_passes=False, use_tc_tiling_on_sc=False),
        scratch_types=[
            pltpu.VMEM((CB * 32, 8), jnp.float32),
            pltpu.VMEM((CB * 32, 8), jnp.float32),
        ],
    )
    def rel(tn_h, out_h, ibuf, obuf):
        wid = lax.axis_index("s") * 2 + lax.axis_index("c")
        iota16 = lax.iota(jnp.int32, 16)
        riota = iota16 >> 3
        ciota = iota16 & 7
        ors = [(2 * iota16 + f) >> 3 for f in (0, 1)]
        ocs = [(2 * iota16 + f) & 7 for f in (0, 1)]

        def chunk_body(ch, carry):
            row0 = (wid * BLK_PER_TILE + ch * CB) * 32
            pltpu.sync_copy(tn_h.at[pl.ds(row0, CB * 32)], ibuf)

            def blk_body(b, c):
                for j in range(8):
                    for f in (0, 1):
                        rsrc = b * 32 + f * 16 + 2 * j + riota
                        v = plsc.load_gather(ibuf, [rsrc, ciota])
                        rdst = b * 32 + 4 * j + ors[f]
                        plsc.store_scatter(obuf, [rdst, ocs[f]], v)
                return c

            lax.fori_loop(0, CB, blk_body, 0)
            pltpu.sync_copy(obuf, out_h.at[pl.ds(row0, CB * 32)])
            return carry

        lax.fori_loop(0, BLK_PER_TILE // CB, chunk_body, 0)

    return rel(tn8)


def _encode(xt, tw8):
    N = xt.shape[1]
    npt = N // NW
    nch = npt // B
    mesh = plsc.VectorSubcoreMesh(core_axis_name="c", subcore_axis_name="s")

    @functools.partial(
        pl.kernel,
        out_type=jax.ShapeDtypeStruct((3, N // 128, 8, 128), jnp.float32),
        mesh=mesh,
        compiler_params=pltpu.CompilerParams(
            needs_layout_passes=False, use_tc_tiling_on_sc=False),
        scratch_types=[
            pltpu.VMEM((3, B), jnp.float32),
            pltpu.VMEM((3, B // 128, 8, 128), jnp.float32),
            pltpu.VMEM((8 * B,), jnp.int32),
            pltpu.VMEM((8 * B,), jnp.int32),
            pltpu.VMEM((8 * B, 8), jnp.float32),
            pltpu.VMEM((8 * B, 8), jnp.float32),
            pltpu.VMEM((TAB0_ROWS * 2 // 8, 8), jnp.float32),
            pltpu.VMEM((TAB1_ROWS * 2 // 8, 8), jnp.float32),
            pltpu.VMEM_SHARED((SPM_TOT, 8), jnp.float32),
            pltpu.SemaphoreType.DMA,
            pltpu.SemaphoreType.DMA,
        ],
    )
    def enc(xt_h, tw8_h, pe_h,
            xb, peb, idxA, idxB, dstA, dstB, tab0, tab1, spm,
            semA, semB):
        wid = lax.axis_index("s") * 2 + lax.axis_index("c")
        idxb = (idxA, idxB)
        dstb = (dstA, dstB)
        sems = (semA, semB)

        pltpu.sync_copy(tw8_h.at[pl.ds(0, TAB0_ROWS * 2 // 8)], tab0)
        pltpu.sync_copy(tw8_h.at[pl.ds(T // 4, TAB1_ROWS * 2 // 8)], tab1)

        @pl.when(lax.axis_index("s") == 0)
        def _stage_spm():
            for li, l in enumerate(SPM_LEVELS):
                pltpu.sync_copy(tw8_h.at[pl.ds(l * (T // 4), SPM_CNT[li])],
                                spm.at[pl.ds(SPM_OFF[l], SPM_CNT[li])])

        plsc.subcore_barrier()

        iota16 = lax.iota(jnp.int32, 16)
        zeros16 = jnp.zeros((16,), jnp.int32)
        ones16 = jnp.full((16,), 1, jnp.int32)

        def coords(g, l):
            res = RES[l]
            resf = float(res)
            s = pl.ds(g * 16, 16)
            fx = xb[0, s] * resf
            fy = xb[1, s] * resf
            fz = xb[2, s] * resf
            ix = jnp.clip(fx.astype(jnp.int32), 0, res - 1)
            iy = jnp.clip(fy.astype(jnp.int32), 0, res - 1)
            iz = jnp.clip(fz.astype(jnp.int32), 0, res - 1)
            return fx, fy, fz, ix, iy, iz

        def corner_rows(l, ix, iy, iz):
            res = RES[l]
            if DENSE[l]:
                stride = res + 1
                ax = (ix, ix + 1)
                ay = (iy * stride, iy * stride + stride)
                az = (iz * (stride * stride), iz * (stride * stride) + stride * stride)
                return [ax[i] + ay[j] + az[k] for (i, j, k) in CORNERS]
            ux, uy, uz = (ix.astype(jnp.uint32), iy.astype(jnp.uint32),
                          iz.astype(jnp.uint32))
            hx = (ux, ux + jnp.uint32(1))
            hy = (uy * jnp.uint32(P1), uy * jnp.uint32(P1) + jnp.uint32(P1))
            hz = (uz * jnp.uint32(P2), uz * jnp.uint32(P2) + jnp.uint32(P2))
            return [((hx[i] ^ hy[j] ^ hz[k]) & jnp.uint32(T - 1)).astype(jnp.int32)
                    for (i, j, k) in CORNERS]

        def weights(fx, fy, fz, ix, iy, iz):
            wx = fx - ix.astype(jnp.float32)
            wy = fy - iy.astype(jnp.float32)
            wz = fz - iz.astype(jnp.float32)
            u = (1.0 - wx, wx)
            v = (1.0 - wy, wy)
            t = (1.0 - wz, wz)
            pxy = [u[i] * v[j] for i in (0, 1) for j in (0, 1)]
            return pxy, t

        def chunk_body(ch, carry):
            base = wid * npt + ch * B
            pltpu.sync_copy(xt_h.at[:, pl.ds(base, B)], xb)

            def norm_body(g, c):
                s = pl.ds(g * 16, 16)
                for d in range(3):
                    xb[d, s] = xb[d, s] * jnp.float32(1.0 / 256.0)
                return c

            lax.fori_loop(0, GROUPS, norm_body, 0)

            def idx_loop(l):
                bi = l % 2
                in_spm = l in SPM_OFF
                rbase = SPM_OFF[l] if in_spm else l * (T // 4)

                def body(g, c, l=l, rbase=rbase, bi=bi):
                    _, _, _, ix, iy, iz = coords(g, l)
                    rows = corner_rows(l, ix, iy, iz)
                    for ci in range(8):
                        idxb[bi][pl.ds(ci * B + g * 16, 16)] = (rows[ci] >> 2) + rbase
                    return c

                lax.fori_loop(0, GROUPS, body, 0)

            def fire(l):
                bi = l % 2
                src = spm if l in SPM_OFF else tw8_h
                return pltpu.async_copy(src.at[idxb[bi]], dstb[bi], sems[bi])

            def acc_loop(l):
                bi = l % 2

                def body(g, c, l=l, bi=bi):
                    fx, fy, fz, ix, iy, iz = coords(g, l)
                    rows = corner_rows(l, ix, iy, iz)
                    pxy, t = weights(fx, fy, fz, ix, iy, iz)
                    prow = g * 16 + iota16
                    acc0 = acc1 = None
                    for ci, (i, j, k) in enumerate(CORNERS):
                        wc = pxy[2 * i + j] * t[k]
                        ocol = (rows[ci] & 3) * 2
                        v0 = plsc.load_gather(dstb[bi], [prow + ci * B, ocol])
                        v1 = plsc.load_gather(dstb[bi], [prow + ci * B, ocol + 1])
                        if acc0 is None:
                            acc0, acc1 = wc * v0, wc * v1
                        else:
                            acc0, acc1 = acc0 + wc * v0, acc1 + wc * v1
                    _pe_store(l, g, acc0, acc1)
                    return c

                lax.fori_loop(0, GROUPS, body, 0)

            def _pe_store(l, g, acc0, acc1):
                ct = g >> 3
                col = (g & 7) * 16
                for f, acc in ((0, acc0), (1, acc1)):
                    r = 2 * l + f
                    peb[r >> 3, ct, r & 7, pl.ds(col, 16)] = acc

            def resident_loop(l, tab):
                def body(g, c, l=l):
                    fx, fy, fz, ix, iy, iz = coords(g, l)
                    rows = corner_rows(l, ix, iy, iz)
                    pxy, t = weights(fx, fy, fz, ix, iy, iz)
                    acc0 = acc1 = None
                    for ci, (i, j, k) in enumerate(CORNERS):
                        wc = pxy[2 * i + j] * t[k]
                        r8 = rows[ci] >> 2
                        oc = (rows[ci] & 3) * 2
                        v0 = plsc.load_gather(tab, [r8, oc])
                        v1 = plsc.load_gather(tab, [r8, oc + 1])
                        if acc0 is None:
                            acc0, acc1 = wc * v0, wc * v1
                        else:
                            acc0, acc1 = acc0 + wc * v0, acc1 + wc * v1
                    _pe_store(l, g, acc0, acc1)
                    return c

                lax.fori_loop(0, GROUPS, body, 0)

            idx_loop(2)
            d_prev = fire(2)
            resident_loop(0, tab0)
            resident_loop(1, tab1)
            for l in range(3, N_LEVELS):
                idx_loop(l)
                d_next = fire(l)
                d_prev.wait()
                acc_loop(l - 1)
                d_prev = d_next
            d_prev.wait()
            acc_loop(N_LEVELS - 1)

            ct0 = base // 128
            for i in range(3):
                pltpu.sync_copy(peb.at[i], pe_h.at[i, pl.ds(ct0, B // 128)])
            return carry

        lax.fori_loop(0, nch, chunk_body, 0)

    return enc(xt, tw8)


def _mlp(pe_t, W0, W1, W2):
    N = pe_t.shape[1]
    BN = 2048
    dn0 = (((0,), (0,)), ((), ()))
    dn = (((1,), (0,)), ((), ()))

    def body(pe_ref, w0_ref, w1_ref, w2_ref, pe_out_ref, z_ref, d_ref):
        pt = pe_ref[...]
        pe_out_ref[...] = pt
        h = jnp.maximum(
            lax.dot_general(pt, w0_ref[...], dn0, preferred_element_type=jnp.float32), 0.0)
        h = jnp.maximum(
            lax.dot_general(h, w1_ref[...], dn, preferred_element_type=jnp.float32), 0.0)
        zt = lax.dot_general(w2_ref[...], h, (((0,), (1,)), ((), ())),
                             preferred_element_type=jnp.float32)
        z_ref[...] = zt
        z0 = zt[0:1, :]
        d_ref[...] = (jnp.maximum(z0, 0.0)
                      + jnp.log1p(jnp.exp(-jnp.abs(z0)))).reshape(-1)

    pe_c, z_t, dens = pl.pallas_call(
        body,
        grid=(N // BN,),
        in_specs=[
            pl.BlockSpec((2 * N_LEVELS, BN), lambda i: (0, i)),
            pl.BlockSpec((2 * N_LEVELS, 64), lambda i: (0, 0)),
            pl.BlockSpec((64, 64), lambda i: (0, 0)),
            pl.BlockSpec((64, 16), lambda i: (0, 0)),
        ],
        out_specs=[
            pl.BlockSpec((2 * N_LEVELS, BN), lambda i: (0, i)),
            pl.BlockSpec((16, BN), lambda i: (0, i)),
            pl.BlockSpec((BN,), lambda i: (i,)),
        ],
        out_shape=[
            jax.ShapeDtypeStruct((2 * N_LEVELS, N), jnp.float32),
            jax.ShapeDtypeStruct((16, N), jnp.float32),
            jax.ShapeDtypeStruct((N,), jnp.float32),
        ],
    )(pe_t, W0, W1, W2)
    return pe_c, z_t, dens


def kernel(x, tables, W0, W1, W2):
    N = x.shape[0]
    tn8 = tables.reshape(N_LEVELS, T // 128, 128, F).transpose(0, 1, 3, 2).reshape(-1, 8)
    tw8 = _relayout(tn8)
    pe4 = _encode(x.T, tw8)
    pe_t = pe4.transpose(0, 2, 1, 3).reshape(2 * N_LEVELS, N)
    pe_c, z_t, dens = _mlp(pe_t, W0, W1, W2)
    return (dens, pe_c.T, z_t.T)
